# Initial kernel scaffold; baseline (speedup 1.0000x reference)
import jax
import jax.numpy as jnp
from jax.experimental import pallas as pl


def kernel(x, cluster_ids, total_buckets, Wq, bq, Wk, bk, Wv, bv, Wg, bg, Wp, bp):
    raise NotImplementedError("write your pallas kernel here")


# speedup vs baseline: 2.8325x; 2.8325x over previous
import functools

import jax
import jax.numpy as jnp
from jax import lax
from jax.experimental import pallas as pl
from jax.experimental.pallas import tpu as pltpu
from jax.experimental.pallas import tpu_sc as plsc

_N = 320000
_C = 128
_TB = 10000
_TBP = 10240
_HEADS = 4
_SCALE = (_C // _HEADS) ** (-0.5)

_NSUB = 16
_NW = 32
_CH = 80
_RPW = _N // _NW
_NCHW = _RPW // _CH
_TROWS = _TBP // _NSUB
_CW = 16

_BLK = 2000

_mesh = plsc.VectorSubcoreMesh(core_axis_name="c", subcore_axis_name="s")


@functools.partial(
    pl.kernel,
    mesh=_mesh,
    out_type=jax.ShapeDtypeStruct((2 * _TBP, _C), jnp.float32),
    scratch_types=[
        pltpu.VMEM((_CH,), jnp.int32),
        pltpu.VMEM((_CH,), jnp.int32),
        pltpu.VMEM((_CH, _C), jnp.float32),
        pltpu.VMEM_SHARED((_TBP, _C), jnp.float32),
        pltpu.SemaphoreType.DMA,
    ],
)
def _seg_sum_sc(x_hbm, ids_hbm, zeros_hbm, ones_hbm, arange_hbm,
                out_hbm, ids_v, iota_v, rows_v, tab_s, sem):
    c = lax.axis_index("c")
    s = lax.axis_index("s")
    t0 = s * _TROWS
    pltpu.sync_copy(zeros_hbm, rows_v)

    def zstep(j, carry):
        o = t0 + j * _CH
        pltpu.sync_copy(arange_hbm.at[pl.ds(o, _CH)], iota_v)
        pltpu.sync_copy(rows_v, tab_s.at[iota_v])
        return carry

    lax.fori_loop(0, _TROWS // _CH, zstep, 0)

    @pl.when(c == 1)
    def _():
        pltpu.sync_copy(ones_hbm, rows_v)

    plsc.subcore_barrier()

    row0 = s * (_N // _NSUB)

    def step(i, carry):
        base = row0 + i * _CH
        pltpu.sync_copy(ids_hbm.at[pl.ds(base, _CH)], ids_v)

        @pl.when(c == 0)
        def _():
            pltpu.sync_copy(x_hbm.at[pl.ds(base, _CH), :], rows_v)

        pltpu.sync_copy(rows_v, tab_s.at[ids_v], add=True)
        return carry

    lax.fori_loop(0, (_N // _NSUB) // _CH, step, 0)
    plsc.subcore_barrier()

    def wstep(j, carry):
        o = t0 + j * _CH
        pltpu.sync_copy(arange_hbm.at[pl.ds(o, _CH)], iota_v)
        pltpu.async_copy(tab_s.at[iota_v], rows_v, sem).wait()
        pltpu.sync_copy(rows_v, out_hbm.at[pl.ds(c * _TBP + o, _CH), :])
        return carry

    lax.fori_loop(0, _TROWS // _CH, wstep, 0)


def _cent_body(xs_ref, cw_ref, xc_ref):
    cnt = jnp.maximum(cw_ref[...][:, 0:1], 1.0)
    xc_ref[...] = xs_ref[...] / cnt


_CB = 2048
_NCB = _TBP // _CB
_cent_call = pl.pallas_call(
    _cent_body,
    grid=(_NCB,),
    in_specs=[
        pl.BlockSpec((_CB, _C), lambda i: (i, 0)),
        pl.BlockSpec((_CB, _C), lambda i: (i + _NCB, 0)),
    ],
    out_specs=pl.BlockSpec((_CB, _C), lambda i: (i, 0)),
    out_shape=jax.ShapeDtypeStruct((_TBP, _C), jnp.float32),
)


@functools.partial(
    pl.kernel,
    mesh=_mesh,
    out_type=jax.ShapeDtypeStruct((_N, _C), jnp.float32),
    scratch_types=[
        pltpu.VMEM((_CH,), jnp.int32),
        pltpu.VMEM((_CH, _C), jnp.float32),
        pltpu.SemaphoreType.DMA,
    ],
)
def _gather_sc(xc_hbm, ids_hbm, ctx_hbm, ids_v, rows_v, sem):
    c = lax.axis_index("c")
    s = lax.axis_index("s")
    row0 = (c * _NSUB + s) * _RPW

    def step(i, carry):
        base = row0 + i * _CH
        pltpu.sync_copy(ids_hbm.at[pl.ds(base, _CH)], ids_v)
        pltpu.async_copy(xc_hbm.at[ids_v], rows_v, sem).wait()
        pltpu.sync_copy(rows_v, ctx_hbm.at[pl.ds(base, _CH), :])
        return carry

    lax.fori_loop(0, _NCHW, step, 0)


def _attn_body(x_ref, xc_ref, wq_ref, bq_ref, wkv_ref, bkv_ref,
               wg_ref, bg_ref, wp_ref, bp_ref, o_ref):
    q = jnp.dot(x_ref[...], wq_ref[...], preferred_element_type=jnp.float32)
    q = q + bq_ref[...]
    kv = jnp.dot(xc_ref[...], wkv_ref[...], preferred_element_type=jnp.float32)
    kv = kv + bkv_ref[...]
    kc = kv[:, :_C]
    vc = kv[:, _C:]
    inter = q * kc * _SCALE
    g = jnp.dot(inter, wg_ref[...], preferred_element_type=jnp.float32)
    attn = jax.nn.sigmoid(g + bg_ref[...])
    o = jnp.dot(attn * vc, wp_ref[...], preferred_element_type=jnp.float32)
    o_ref[...] = o + bp_ref[...]


_attn_call = pl.pallas_call(
    _attn_body,
    grid=(_N // _BLK,),
    in_specs=[
        pl.BlockSpec((_BLK, _C), lambda i: (i, 0)),
        pl.BlockSpec((_BLK, _C), lambda i: (i, 0)),
        pl.BlockSpec((_C, _C), lambda i: (0, 0)),
        pl.BlockSpec((1, _C), lambda i: (0, 0)),
        pl.BlockSpec((_C, 2 * _C), lambda i: (0, 0)),
        pl.BlockSpec((1, 2 * _C), lambda i: (0, 0)),
        pl.BlockSpec((_C, _C), lambda i: (0, 0)),
        pl.BlockSpec((1, _C), lambda i: (0, 0)),
        pl.BlockSpec((_C, _C), lambda i: (0, 0)),
        pl.BlockSpec((1, _C), lambda i: (0, 0)),
    ],
    out_specs=pl.BlockSpec((_BLK, _C), lambda i: (i, 0)),
    out_shape=jax.ShapeDtypeStruct((_N, _C), jnp.float32),
)


def kernel(x, cluster_ids, total_buckets, Wq, bq, Wk, bk, Wv, bv, Wg, bg, Wp, bp):
    ids = jnp.minimum(cluster_ids, total_buckets - 1).astype(jnp.int32)

    zeros = jnp.zeros((_CH, _C), jnp.float32)
    ones = jnp.ones((_CH, _C), jnp.float32)
    arange = jnp.arange(_TBP, dtype=jnp.int32)
    tabs = _seg_sum_sc(x, ids, zeros, ones, arange)

    xcent = _cent_call(tabs, tabs)

    xctx = _gather_sc(xcent, ids)

    wkv = jnp.concatenate([Wk, Wv], axis=1)
    bkv = jnp.concatenate([bk, bv]).reshape(1, 2 * _C)
    return _attn_call(x, xctx,
                      Wq, bq.reshape(1, _C), wkv, bkv,
                      Wg, bg.reshape(1, _C),
                      Wp, bp.reshape(1, _C))


# speedup vs baseline: 3.8914x; 1.3738x over previous
import functools

import jax
import jax.numpy as jnp
from jax import lax
from jax.experimental import pallas as pl
from jax.experimental.pallas import tpu as pltpu
from jax.experimental.pallas import tpu_sc as plsc

_N = 320000
_C = 128
_TB = 10000
_TBP = 10240
_HEADS = 4
_SCALE = (_C // _HEADS) ** (-0.5)

_NSUB = 16
_NW = 32
_CH = 80
_RPW = _N // _NW
_NCHW = _RPW // _CH
_TROWS = _TBP // _NSUB
_CW = 16

_BLK = 2000

_mesh = plsc.VectorSubcoreMesh(core_axis_name="c", subcore_axis_name="s")


_SSLAB = 160
_SSPC = _SSLAB // _CH
_TS_SLABS = (_N // _NSUB) // _SSLAB
_IDG = 8


@functools.partial(
    pl.kernel,
    mesh=_mesh,
    out_type=jax.ShapeDtypeStruct((2 * _TBP, _C), jnp.float32),
    scratch_types=[
        pltpu.VMEM((_IDG, _CH), jnp.int32),
        pltpu.VMEM((_CH,), jnp.int32),
        pltpu.VMEM((_SSLAB, _C), jnp.float32),
        pltpu.VMEM_SHARED((_TBP, _C), jnp.float32),
        pltpu.SemaphoreType.DMA,
    ],
)
def _seg_sum_sc(x_hbm, ids_hbm, zeros_hbm, ones_hbm, arange_hbm,
                out_hbm, ids_vm, iota_v, slab_v, tab_s, sem):
    c = lax.axis_index("c")
    s = lax.axis_index("s")
    t0 = s * _TROWS
    pltpu.sync_copy(zeros_hbm, slab_v.at[pl.ds(0, _CH), :])

    def zstep(j, carry):
        o = t0 + j * _CH
        pltpu.sync_copy(arange_hbm.at[pl.ds(o, _CH)], iota_v)
        pltpu.sync_copy(slab_v.at[pl.ds(0, _CH), :], tab_s.at[iota_v])
        return carry

    lax.fori_loop(0, _TROWS // _CH, zstep, 0)

    @pl.when(c == 1)
    def _():
        pltpu.sync_copy(ones_hbm, slab_v.at[pl.ds(0, _CH), :])

    plsc.subcore_barrier()

    row0 = s * (_N // _NSUB)
    ig0 = s * _TS_SLABS

    def step(i2, carry):
        pltpu.sync_copy(ids_hbm.at[pl.ds((ig0 + i2) * _IDG, _IDG), :], ids_vm)

        @pl.when(c == 0)
        def _():
            pltpu.sync_copy(x_hbm.at[pl.ds(row0 + i2 * _SSLAB, _SSLAB), :],
                            slab_v)

        for j in range(_SSPC):
            idx = ids_vm.at[j]

            @pl.when(c == 0)
            def _():
                pltpu.sync_copy(slab_v.at[pl.ds(j * _CH, _CH), :],
                                tab_s.at[idx], add=True)

            @pl.when(c == 1)
            def _():
                pltpu.sync_copy(slab_v.at[pl.ds(0, _CH), :],
                                tab_s.at[idx], add=True)

        return carry

    lax.fori_loop(0, _TS_SLABS, step, 0)
    plsc.subcore_barrier()

    def wstep(j, carry):
        o = t0 + j * _CH
        pltpu.sync_copy(arange_hbm.at[pl.ds(o, _CH)], iota_v)
        pltpu.async_copy(tab_s.at[iota_v], slab_v.at[pl.ds(0, _CH), :],
                         sem).wait()
        pltpu.sync_copy(slab_v.at[pl.ds(0, _CH), :],
                        out_hbm.at[pl.ds(c * _TBP + o, _CH), :])
        return carry

    lax.fori_loop(0, _TROWS // _CH, wstep, 0)


def _cent_body(xs_ref, cw_ref, xc_ref):
    cnt = jnp.maximum(cw_ref[...][:, 0:1], 1.0)
    xc_ref[...] = xs_ref[...] / cnt


_CB = 2048
_NCB = _TBP // _CB
_cent_call = pl.pallas_call(
    _cent_body,
    grid=(_NCB,),
    in_specs=[
        pl.BlockSpec((_CB, _C), lambda i: (i, 0)),
        pl.BlockSpec((_CB, _C), lambda i: (i + _NCB, 0)),
    ],
    out_specs=pl.BlockSpec((_CB, _C), lambda i: (i, 0)),
    out_shape=jax.ShapeDtypeStruct((_TBP, _C), jnp.float32),
)


_GW_CH = _RPW // _CH
_GW_PAD = 128
_GSLAB = 400
_GSPC = _GSLAB // _CH


@functools.partial(
    pl.kernel,
    mesh=_mesh,
    out_type=jax.ShapeDtypeStruct((_N, _C), jnp.float32),
    scratch_types=[
        pltpu.VMEM((_GW_PAD, _CH), jnp.int32),
        pltpu.VMEM((_GSLAB, _C), jnp.float32),
        pltpu.SemaphoreType.DMA,
    ],
)
def _gather_sc(xc_hbm, ids_hbm, ctx_hbm, ids_vm, slab_v, sem):
    c = lax.axis_index("c")
    s = lax.axis_index("s")
    w = c * _NSUB + s
    row0 = w * _RPW
    pltpu.sync_copy(ids_hbm.at[pl.ds(w * _GW_PAD, _GW_PAD), :], ids_vm)

    def step(i2, carry):
        handles = []
        for j in range(_GSPC):
            handles.append(pltpu.async_copy(
                xc_hbm.at[ids_vm.at[i2 * _GSPC + j]],
                slab_v.at[pl.ds(j * _CH, _CH), :], sem))
        for h in handles:
            h.wait()
        pltpu.sync_copy(slab_v, ctx_hbm.at[pl.ds(row0 + i2 * _GSLAB, _GSLAB), :])
        return carry

    lax.fori_loop(0, _GW_CH // _GSPC, step, 0)


def _attn_body(x_ref, xc_ref, wq_ref, bq_ref, wkv_ref, bkv_ref,
               wg_ref, bg_ref, wp_ref, bp_ref, o_ref):
    q = jnp.dot(x_ref[...], wq_ref[...], preferred_element_type=jnp.float32)
    q = q + bq_ref[...]
    kv = jnp.dot(xc_ref[...], wkv_ref[...], preferred_element_type=jnp.float32)
    kv = kv + bkv_ref[...]
    kc = kv[:, :_C]
    vc = kv[:, _C:]
    inter = q * kc * _SCALE
    g = jnp.dot(inter, wg_ref[...], preferred_element_type=jnp.float32)
    attn = jax.nn.sigmoid(g + bg_ref[...])
    o = jnp.dot(attn * vc, wp_ref[...], preferred_element_type=jnp.float32)
    o_ref[...] = o + bp_ref[...]


_attn_call = pl.pallas_call(
    _attn_body,
    grid=(_N // _BLK,),
    in_specs=[
        pl.BlockSpec((_BLK, _C), lambda i: (i, 0)),
        pl.BlockSpec((_BLK, _C), lambda i: (i, 0)),
        pl.BlockSpec((_C, _C), lambda i: (0, 0)),
        pl.BlockSpec((1, _C), lambda i: (0, 0)),
        pl.BlockSpec((_C, 2 * _C), lambda i: (0, 0)),
        pl.BlockSpec((1, 2 * _C), lambda i: (0, 0)),
        pl.BlockSpec((_C, _C), lambda i: (0, 0)),
        pl.BlockSpec((1, _C), lambda i: (0, 0)),
        pl.BlockSpec((_C, _C), lambda i: (0, 0)),
        pl.BlockSpec((1, _C), lambda i: (0, 0)),
    ],
    out_specs=pl.BlockSpec((_BLK, _C), lambda i: (i, 0)),
    out_shape=jax.ShapeDtypeStruct((_N, _C), jnp.float32),
)


def kernel(x, cluster_ids, total_buckets, Wq, bq, Wk, bk, Wv, bv, Wg, bg, Wp, bp):
    ids = jnp.minimum(cluster_ids, total_buckets - 1).astype(jnp.int32)
    ids2d = ids.reshape(_N // _CH, _CH)
    idsA = jnp.pad(ids2d.reshape(_NSUB * _TS_SLABS, _SSPC, _CH),
                   ((0, 0), (0, _IDG - _SSPC), (0, 0))).reshape(-1, _CH)
    idsB = jnp.pad(ids2d.reshape(_NW, _GW_CH, _CH),
                   ((0, 0), (0, _GW_PAD - _GW_CH), (0, 0))).reshape(-1, _CH)

    zeros = jnp.zeros((_CH, _C), jnp.float32)
    ones = jnp.ones((_CH, _C), jnp.float32)
    arange = jnp.arange(_TBP, dtype=jnp.int32)
    tabs = _seg_sum_sc(x, idsA, zeros, ones, arange)

    xcent = _cent_call(tabs, tabs)

    xctx = _gather_sc(xcent, idsB)

    wkv = jnp.concatenate([Wk, Wv], axis=1)
    bkv = jnp.concatenate([bk, bv]).reshape(1, 2 * _C)
    return _attn_call(x, xctx,
                      Wq, bq.reshape(1, _C), wkv, bkv,
                      Wg, bg.reshape(1, _C),
                      Wp, bp.reshape(1, _C))


# speedup vs baseline: 3.9475x; 1.0144x over previous
import functools

import jax
import jax.numpy as jnp
from jax import lax
from jax.experimental import pallas as pl
from jax.experimental.pallas import tpu as pltpu
from jax.experimental.pallas import tpu_sc as plsc

_N = 320000
_C = 128
_TB = 10000
_TBP = 10240
_HEADS = 4
_SCALE = (_C // _HEADS) ** (-0.5)

_NSUB = 16
_NW = 32
_CH = 80
_RPW = _N // _NW
_NCHW = _RPW // _CH
_TROWS = _TBP // _NSUB
_CW = 16

_BLK = 2000

_mesh = plsc.VectorSubcoreMesh(core_axis_name="c", subcore_axis_name="s")


_SSLAB = 160
_SSPC = _SSLAB // _CH
_TS_SLABS = (_N // _NSUB) // _SSLAB
_IDG = 8


@functools.partial(
    pl.kernel,
    mesh=_mesh,
    out_type=jax.ShapeDtypeStruct((2 * _TBP, _C), jnp.float32),
    scratch_types=[
        pltpu.VMEM((_IDG, _CH), jnp.int32),
        pltpu.VMEM((_CH,), jnp.int32),
        pltpu.VMEM((_SSLAB, _C), jnp.float32),
        pltpu.VMEM_SHARED((_TBP, _C), jnp.float32),
        pltpu.SemaphoreType.DMA,
    ],
)
def _seg_sum_sc(x_hbm, ids_hbm, zeros_hbm, ones_hbm, arange_hbm,
                out_hbm, ids_vm, iota_v, slab_v, tab_s, sem):
    c = lax.axis_index("c")
    s = lax.axis_index("s")
    t0 = s * _TROWS
    pltpu.sync_copy(zeros_hbm, slab_v.at[pl.ds(0, _CH), :])

    def zstep(j, carry):
        o = t0 + j * _CH
        pltpu.sync_copy(arange_hbm.at[pl.ds(o, _CH)], iota_v)
        pltpu.sync_copy(slab_v.at[pl.ds(0, _CH), :], tab_s.at[iota_v])
        return carry

    lax.fori_loop(0, _TROWS // _CH, zstep, 0)

    @pl.when(c == 1)
    def _():
        pltpu.sync_copy(ones_hbm, slab_v.at[pl.ds(0, _CH), :])

    plsc.subcore_barrier()

    row0 = s * (_N // _NSUB)
    ig0 = s * _TS_SLABS

    def step(i2, carry):
        pltpu.sync_copy(ids_hbm.at[pl.ds((ig0 + i2) * _IDG, _IDG), :], ids_vm)

        @pl.when(c == 0)
        def _():
            pltpu.sync_copy(x_hbm.at[pl.ds(row0 + i2 * _SSLAB, _SSLAB), :],
                            slab_v)

        for j in range(_SSPC):
            idx = ids_vm.at[j]

            @pl.when(c == 0)
            def _():
                pltpu.sync_copy(slab_v.at[pl.ds(j * _CH, _CH), :],
                                tab_s.at[idx], add=True)

            @pl.when(c == 1)
            def _():
                pltpu.sync_copy(slab_v.at[pl.ds(0, _CH), :],
                                tab_s.at[idx], add=True)

        return carry

    lax.fori_loop(0, _TS_SLABS, step, 0)
    plsc.subcore_barrier()

    def wstep(j, carry):
        o = t0 + j * _CH
        pltpu.sync_copy(arange_hbm.at[pl.ds(o, _CH)], iota_v)
        pltpu.async_copy(tab_s.at[iota_v], slab_v.at[pl.ds(0, _CH), :],
                         sem).wait()
        pltpu.sync_copy(slab_v.at[pl.ds(0, _CH), :],
                        out_hbm.at[pl.ds(c * _TBP + o, _CH), :])
        return carry

    lax.fori_loop(0, _TROWS // _CH, wstep, 0)


def _cent_body(xs_ref, cw_ref, xc_ref):
    cnt = jnp.maximum(cw_ref[...][:, 0:1], 1.0)
    xc_ref[...] = xs_ref[...] / cnt


_CB = 2048
_NCB = _TBP // _CB
_cent_call = pl.pallas_call(
    _cent_body,
    grid=(_NCB,),
    in_specs=[
        pl.BlockSpec((_CB, _C), lambda i: (i, 0)),
        pl.BlockSpec((_CB, _C), lambda i: (i + _NCB, 0)),
    ],
    out_specs=pl.BlockSpec((_CB, _C), lambda i: (i, 0)),
    out_shape=jax.ShapeDtypeStruct((_TBP, _C), jnp.float32),
)


_GW_CH = _RPW // _CH
_GW_PAD = 128
_GSLAB = 400
_GSPC = _GSLAB // _CH


@functools.partial(
    pl.kernel,
    mesh=_mesh,
    out_type=jax.ShapeDtypeStruct((_N, _C), jnp.float32),
    scratch_types=[
        pltpu.VMEM((_GW_PAD, _CH), jnp.int32),
        pltpu.VMEM((_GSLAB, _C), jnp.float32),
        pltpu.VMEM((_GSLAB, _C), jnp.float32),
        pltpu.SemaphoreType.DMA,
        pltpu.SemaphoreType.DMA,
        pltpu.SemaphoreType.DMA,
    ],
)
def _gather_sc(xc_hbm, ids_hbm, ctx_hbm, ids_vm, slab_a, slab_b, semg, semw_a, semw_b):
    c = lax.axis_index("c")
    s = lax.axis_index("s")
    w = c * _NSUB + s
    row0 = w * _RPW
    pltpu.sync_copy(ids_hbm.at[pl.ds(w * _GW_PAD, _GW_PAD), :], ids_vm)

    slabs = (slab_a, slab_b)
    semws = (semw_a, semw_b)
    pending = [None, None]
    for i in range(_GW_CH // _GSPC):
        b = i % 2
        if pending[b] is not None:
            pending[b].wait()
        gathers = []
        for j in range(_GSPC):
            gathers.append(pltpu.async_copy(
                xc_hbm.at[ids_vm.at[i * _GSPC + j]],
                slabs[b].at[pl.ds(j * _CH, _CH), :], semg))
        for h in gathers:
            h.wait()
        pending[b] = pltpu.async_copy(
            slabs[b], ctx_hbm.at[pl.ds(row0 + i * _GSLAB, _GSLAB), :], semws[b])
    for h in pending:
        if h is not None:
            h.wait()


def _attn_body(x_ref, xc_ref, wq_ref, bq_ref, wkv_ref, bkv_ref,
               wg_ref, bg_ref, wp_ref, bp_ref, o_ref):
    q = jnp.dot(x_ref[...], wq_ref[...], preferred_element_type=jnp.float32)
    q = q + bq_ref[...]
    kv = jnp.dot(xc_ref[...], wkv_ref[...], preferred_element_type=jnp.float32)
    kv = kv + bkv_ref[...]
    kc = kv[:, :_C]
    vc = kv[:, _C:]
    inter = q * kc * _SCALE
    g = jnp.dot(inter, wg_ref[...], preferred_element_type=jnp.float32)
    attn = jax.nn.sigmoid(g + bg_ref[...])
    o = jnp.dot(attn * vc, wp_ref[...], preferred_element_type=jnp.float32)
    o_ref[...] = o + bp_ref[...]


_attn_call = pl.pallas_call(
    _attn_body,
    grid=(_N // _BLK,),
    in_specs=[
        pl.BlockSpec((_BLK, _C), lambda i: (i, 0)),
        pl.BlockSpec((_BLK, _C), lambda i: (i, 0)),
        pl.BlockSpec((_C, _C), lambda i: (0, 0)),
        pl.BlockSpec((1, _C), lambda i: (0, 0)),
        pl.BlockSpec((_C, 2 * _C), lambda i: (0, 0)),
        pl.BlockSpec((1, 2 * _C), lambda i: (0, 0)),
        pl.BlockSpec((_C, _C), lambda i: (0, 0)),
        pl.BlockSpec((1, _C), lambda i: (0, 0)),
        pl.BlockSpec((_C, _C), lambda i: (0, 0)),
        pl.BlockSpec((1, _C), lambda i: (0, 0)),
    ],
    out_specs=pl.BlockSpec((_BLK, _C), lambda i: (i, 0)),
    out_shape=jax.ShapeDtypeStruct((_N, _C), jnp.float32),
)


def kernel(x, cluster_ids, total_buckets, Wq, bq, Wk, bk, Wv, bv, Wg, bg, Wp, bp):
    ids = jnp.minimum(cluster_ids, total_buckets - 1).astype(jnp.int32)
    ids2d = ids.reshape(_N // _CH, _CH)
    idsA = jnp.pad(ids2d.reshape(_NSUB * _TS_SLABS, _SSPC, _CH),
                   ((0, 0), (0, _IDG - _SSPC), (0, 0))).reshape(-1, _CH)
    idsB = jnp.pad(ids2d.reshape(_NW, _GW_CH, _CH),
                   ((0, 0), (0, _GW_PAD - _GW_CH), (0, 0))).reshape(-1, _CH)

    zeros = jnp.zeros((_CH, _C), jnp.float32)
    ones = jnp.ones((_CH, _C), jnp.float32)
    arange = jnp.arange(_TBP, dtype=jnp.int32)
    tabs = _seg_sum_sc(x, idsA, zeros, ones, arange)

    xcent = _cent_call(tabs, tabs)

    xctx = _gather_sc(xcent, idsB)

    wkv = jnp.concatenate([Wk, Wv], axis=1)
    bkv = jnp.concatenate([bk, bv]).reshape(1, 2 * _C)
    return _attn_call(x, xctx,
                      Wq, bq.reshape(1, _C), wkv, bkv,
                      Wg, bg.reshape(1, _C),
                      Wp, bp.reshape(1, _C))


# speedup vs baseline: 4.6843x; 1.1867x over previous
import functools

import jax
import jax.numpy as jnp
from jax import lax
from jax.experimental import pallas as pl
from jax.experimental.pallas import tpu as pltpu
from jax.experimental.pallas import tpu_sc as plsc

_N = 320000
_C = 128
_TB = 10000
_TBP = 10240
_HEADS = 4
_SCALE = (_C // _HEADS) ** (-0.5)

_NSUB = 16
_NW = 32
_CH = 80
_RPW = _N // _NW
_NCHW = _RPW // _CH
_TROWS = _TBP // _NSUB
_CW = 16

_BLK = 2000

_mesh = plsc.VectorSubcoreMesh(core_axis_name="c", subcore_axis_name="s")


_SSLAB = 160
_SSPC = _SSLAB // _CH
_TS_SLABS = (_N // _NSUB) // _SSLAB
_TS_CHUNKS = (_N // _NSUB) // _CH
_IDG = 8


@functools.partial(
    pl.kernel,
    mesh=_mesh,
    out_type=jax.ShapeDtypeStruct((2 * _TBP, _C), jnp.float32),
    scratch_types=[
        pltpu.VMEM((_IDG, _CH), jnp.int32),
        pltpu.VMEM((_IDG, _CH), jnp.int32),
        pltpu.VMEM((_CH,), jnp.int32),
        pltpu.VMEM((_CH, _C), jnp.float32),
        pltpu.VMEM((_CH, _C), jnp.float32),
        pltpu.VMEM((_CH, _C), jnp.float32),
        pltpu.VMEM_SHARED((_TBP, _C), jnp.float32),
        pltpu.SemaphoreType.DMA,
        pltpu.SemaphoreType.DMA,
        pltpu.SemaphoreType.DMA,
        pltpu.SemaphoreType.DMA,
        pltpu.SemaphoreType.DMA,
    ],
)
def _seg_sum_sc(x_hbm, ids_hbm, zeros_hbm, ones_hbm, arange_hbm,
                out_hbm, ids_a, ids_b, iota_v, x_a, x_b, ones_v, tab_s,
                semx_a, semx_b, semi_a, semi_b, sem):
    c = lax.axis_index("c")
    s = lax.axis_index("s")
    t0 = s * _TROWS
    pltpu.sync_copy(zeros_hbm, ones_v)

    def zstep(j, carry):
        o = t0 + j * _CH
        pltpu.sync_copy(arange_hbm.at[pl.ds(o, _CH)], iota_v)
        pltpu.sync_copy(ones_v, tab_s.at[iota_v])
        return carry

    lax.fori_loop(0, _TROWS // _CH, zstep, 0)

    @pl.when(c == 1)
    def _():
        pltpu.sync_copy(ones_hbm, ones_v)

    plsc.subcore_barrier()

    row0 = s * (_N // _NSUB)
    ig0 = s * _TS_SLABS
    idsbuf = (ids_a, ids_b)
    xbuf = (x_a, x_b)
    semx = (semx_a, semx_b)
    semi = (semi_a, semi_b)

    for k in range(2):
        pltpu.async_copy(ids_hbm.at[pl.ds(ig0 * _IDG, _IDG), :],
                         idsbuf[k], semi[k])

        @pl.when(c == 0)
        def _():
            pltpu.async_copy(x_hbm.at[pl.ds(row0 + k * _CH, _CH), :],
                             xbuf[k], semx[k])

    def pair(i2, carry):
        for k in range(2):
            pltpu.make_async_copy(ids_hbm.at[pl.ds(0, _IDG), :],
                                  idsbuf[k], semi[k]).wait()

            @pl.when(c == 0)
            def _():
                pltpu.make_async_copy(x_hbm.at[pl.ds(0, _CH), :],
                                      xbuf[k], semx[k]).wait()
                pltpu.sync_copy(xbuf[k], tab_s.at[idsbuf[k].at[k]], add=True)

            @pl.when(c == 1)
            def _():
                pltpu.sync_copy(ones_v, tab_s.at[idsbuf[k].at[k]], add=True)

            nxt = lax.min(2 * i2 + k + 2, _TS_CHUNKS - 1)
            grp = lax.min(i2 + 1, _TS_SLABS - 1)
            pltpu.async_copy(ids_hbm.at[pl.ds((ig0 + grp) * _IDG, _IDG), :],
                             idsbuf[k], semi[k])

            @pl.when(c == 0)
            def _():
                pltpu.async_copy(x_hbm.at[pl.ds(row0 + nxt * _CH, _CH), :],
                                 xbuf[k], semx[k])

        return carry

    lax.fori_loop(0, _TS_CHUNKS // 2, pair, 0)
    for k in range(2):
        pltpu.make_async_copy(ids_hbm.at[pl.ds(0, _IDG), :],
                              idsbuf[k], semi[k]).wait()

        @pl.when(c == 0)
        def _():
            pltpu.make_async_copy(x_hbm.at[pl.ds(0, _CH), :],
                                  xbuf[k], semx[k]).wait()

    plsc.subcore_barrier()

    def wstep(j, carry):
        o = t0 + j * _CH
        pltpu.sync_copy(arange_hbm.at[pl.ds(o, _CH)], iota_v)
        pltpu.async_copy(tab_s.at[iota_v], ones_v, sem).wait()
        pltpu.sync_copy(ones_v, out_hbm.at[pl.ds(c * _TBP + o, _CH), :])
        return carry

    lax.fori_loop(0, _TROWS // _CH, wstep, 0)


def _cent_body(xs_ref, cw_ref, xc_ref):
    cnt = jnp.maximum(cw_ref[...][:, 0:1], 1.0)
    xc_ref[...] = xs_ref[...] / cnt


_CB = 2048
_NCB = _TBP // _CB
_cent_call = pl.pallas_call(
    _cent_body,
    grid=(_NCB,),
    in_specs=[
        pl.BlockSpec((_CB, _C), lambda i: (i, 0)),
        pl.BlockSpec((_CB, _C), lambda i: (i + _NCB, 0)),
    ],
    out_specs=pl.BlockSpec((_CB, _C), lambda i: (i, 0)),
    out_shape=jax.ShapeDtypeStruct((_TBP, _C), jnp.float32),
)


_GW_CH = _RPW // _CH
_GW_PAD = 128
_GSLAB = 400
_GSPC = _GSLAB // _CH


@functools.partial(
    pl.kernel,
    mesh=_mesh,
    out_type=jax.ShapeDtypeStruct((_N, _C), jnp.float32),
    scratch_types=[
        pltpu.VMEM((_GW_PAD, _CH), jnp.int32),
        pltpu.VMEM((_GSLAB, _C), jnp.float32),
        pltpu.VMEM((_GSLAB, _C), jnp.float32),
        pltpu.SemaphoreType.DMA,
        pltpu.SemaphoreType.DMA,
        pltpu.SemaphoreType.DMA,
    ],
)
def _gather_sc(xc_hbm, ids_hbm, ctx_hbm, ids_vm, slab_a, slab_b, semg, semw_a, semw_b):
    c = lax.axis_index("c")
    s = lax.axis_index("s")
    w = c * _NSUB + s
    row0 = w * _RPW
    pltpu.sync_copy(ids_hbm.at[pl.ds(w * _GW_PAD, _GW_PAD), :], ids_vm)

    slabs = (slab_a, slab_b)
    semws = (semw_a, semw_b)
    pending = [None, None]
    for i in range(_GW_CH // _GSPC):
        b = i % 2
        if pending[b] is not None:
            pending[b].wait()
        gathers = []
        for j in range(_GSPC):
            gathers.append(pltpu.async_copy(
                xc_hbm.at[ids_vm.at[i * _GSPC + j]],
                slabs[b].at[pl.ds(j * _CH, _CH), :], semg))
        for h in gathers:
            h.wait()
        pending[b] = pltpu.async_copy(
            slabs[b], ctx_hbm.at[pl.ds(row0 + i * _GSLAB, _GSLAB), :], semws[b])
    for h in pending:
        if h is not None:
            h.wait()


def _attn_body(x_ref, xc_ref, wq_ref, bq_ref, wkv_ref, bkv_ref,
               wg_ref, bg_ref, wp_ref, bp_ref, o_ref):
    q = jnp.dot(x_ref[...], wq_ref[...], preferred_element_type=jnp.float32)
    q = q + bq_ref[...]
    kv = jnp.dot(xc_ref[...], wkv_ref[...], preferred_element_type=jnp.float32)
    kv = kv + bkv_ref[...]
    kc = kv[:, :_C]
    vc = kv[:, _C:]
    inter = q * kc * _SCALE
    g = jnp.dot(inter, wg_ref[...], preferred_element_type=jnp.float32)
    attn = jax.nn.sigmoid(g + bg_ref[...])
    o = jnp.dot(attn * vc, wp_ref[...], preferred_element_type=jnp.float32)
    o_ref[...] = o + bp_ref[...]


_attn_call = pl.pallas_call(
    _attn_body,
    grid=(_N // _BLK,),
    in_specs=[
        pl.BlockSpec((_BLK, _C), lambda i: (i, 0)),
        pl.BlockSpec((_BLK, _C), lambda i: (i, 0)),
        pl.BlockSpec((_C, _C), lambda i: (0, 0)),
        pl.BlockSpec((1, _C), lambda i: (0, 0)),
        pl.BlockSpec((_C, 2 * _C), lambda i: (0, 0)),
        pl.BlockSpec((1, 2 * _C), lambda i: (0, 0)),
        pl.BlockSpec((_C, _C), lambda i: (0, 0)),
        pl.BlockSpec((1, _C), lambda i: (0, 0)),
        pl.BlockSpec((_C, _C), lambda i: (0, 0)),
        pl.BlockSpec((1, _C), lambda i: (0, 0)),
    ],
    out_specs=pl.BlockSpec((_BLK, _C), lambda i: (i, 0)),
    out_shape=jax.ShapeDtypeStruct((_N, _C), jnp.float32),
)


def kernel(x, cluster_ids, total_buckets, Wq, bq, Wk, bk, Wv, bv, Wg, bg, Wp, bp):
    ids = jnp.minimum(cluster_ids, total_buckets - 1).astype(jnp.int32)
    ids2d = ids.reshape(_N // _CH, _CH)
    idsA = jnp.pad(ids2d.reshape(_NSUB * _TS_SLABS, _SSPC, _CH),
                   ((0, 0), (0, _IDG - _SSPC), (0, 0))).reshape(-1, _CH)
    idsB = jnp.pad(ids2d.reshape(_NW, _GW_CH, _CH),
                   ((0, 0), (0, _GW_PAD - _GW_CH), (0, 0))).reshape(-1, _CH)

    zeros = jnp.zeros((_CH, _C), jnp.float32)
    ones = jnp.ones((_CH, _C), jnp.float32)
    arange = jnp.arange(_TBP, dtype=jnp.int32)
    tabs = _seg_sum_sc(x, idsA, zeros, ones, arange)

    xcent = _cent_call(tabs, tabs)

    xctx = _gather_sc(xcent, idsB)

    wkv = jnp.concatenate([Wk, Wv], axis=1)
    bkv = jnp.concatenate([bk, bv]).reshape(1, 2 * _C)
    return _attn_call(x, xctx,
                      Wq, bq.reshape(1, _C), wkv, bkv,
                      Wg, bg.reshape(1, _C),
                      Wp, bp.reshape(1, _C))
